# baseline (speedup 1.0000x reference)
import functools

import jax
import jax.numpy as jnp
from jax import lax
from jax.experimental import pallas as pl
from jax.experimental.pallas import tpu as pltpu
from jax.experimental.pallas import tpu_sc as plsc

_VOCAB = 100000
_NUM_CLASSES = 2
_BATCH = 1024
_SEQ = 200
_N_SUBCORES = 16
_ROWS_PER = _BATCH // _N_SUBCORES
_IDS_PER = _ROWS_PER * _SEQ
_LANES = 16
_GROUPS = _ROWS_PER // _LANES
_WORDS = _VOCAB // 2
_N_CHUNKS = 10
_CHUNK = _WORDS // _N_CHUNKS


def _bow_body(ids_hbm, w_hbm, out_hbm, table_v, ids_v, out_v, sem_w, sem_i):
    cls = lax.axis_index("c")
    sub = lax.axis_index("s")
    rowbase = sub * _ROWS_PER

    w_base = pl.multiple_of(cls * _WORDS, 8)
    start = lax.rem(sub, _N_CHUNKS)
    cps = []
    for k in range(_N_CHUNKS):
        sl = lax.rem(start + k, _N_CHUNKS) * _CHUNK
        sl = pl.multiple_of(sl, 8)
        cps.append(pltpu.async_copy(w_hbm.at[pl.ds(w_base + sl, _CHUNK)],
                                    table_v.at[pl.ds(sl, _CHUNK)], sem_w))
    ids_off = pl.multiple_of(sub * _IDS_PER, 8)
    cp_i = pltpu.async_copy(ids_hbm.at[pl.ds(ids_off, _IDS_PER)], ids_v, sem_i)
    for cp in cps:
        cp.wait()
    cp_i.wait()

    lane = lax.iota(jnp.int32, _LANES)
    head = plsc.bitcast(table_v[pl.ds(0, _LANES)], jnp.int32)
    patched = jnp.bitwise_and(head, jnp.int32(-65536))
    table_v[pl.ds(0, _LANES)] = plsc.bitcast(
        jnp.where(lane == 0, patched, head), jnp.float32)

    bases = [(g * _LANES + lane) * _SEQ for g in range(_GROUPS)]
    zero = jnp.zeros((_LANES,), jnp.float32)

    def extract(tid):
        hi = tid >= _WORDS
        widx = jnp.where(hi, tid - _WORDS, tid)
        word = plsc.bitcast(plsc.load_gather(table_v, [widx]), jnp.int32)
        sh = jnp.where(hi, jnp.int32(16), jnp.int32(0))
        bits = lax.shift_left(lax.shift_right_logical(word, sh), 16)
        return plsc.bitcast(bits, jnp.float32)

    def step(t, accs):
        ids16 = [plsc.load_gather(ids_v, [bases[g] + t])
                 for g in range(_GROUPS)]
        vals = [extract(ix) for ix in ids16]
        return tuple(accs[g] + vals[g] for g in range(_GROUPS))

    accs = lax.fori_loop(0, _SEQ, step, (zero,) * _GROUPS)
    for g in range(_GROUPS):
        out_v[pl.ds(g * _LANES, _LANES)] = accs[g]

    out_off = pl.multiple_of(cls * _BATCH + rowbase, 8)
    pltpu.sync_copy(out_v, out_hbm.at[pl.ds(out_off, _ROWS_PER)])


@jax.jit
def _bow_sc(ids_flat, w_flat):
    mesh = plsc.VectorSubcoreMesh(core_axis_name="c", subcore_axis_name="s")
    f = functools.partial(
        pl.kernel,
        mesh=mesh,
        compiler_params=pltpu.CompilerParams(needs_layout_passes=False),
        out_type=jax.ShapeDtypeStruct((_NUM_CLASSES * _BATCH,), jnp.float32),
        scratch_types=[
            pltpu.VMEM((_WORDS,), jnp.float32),
            pltpu.VMEM((_IDS_PER,), jnp.int32),
            pltpu.VMEM((_ROWS_PER,), jnp.float32),
            pltpu.SemaphoreType.DMA,
            pltpu.SemaphoreType.DMA,
        ],
    )(_bow_body)
    return f(ids_flat, w_flat)


def kernel(input_ids, W, b):
    ids_flat = input_ids.astype(jnp.int32).reshape(-1)
    w32 = jax.lax.bitcast_convert_type(W.astype(jnp.float32), jnp.int32)
    rne = jax.lax.shift_right_logical(
        w32 + 0x7FFF + jnp.bitwise_and(jax.lax.shift_right_logical(w32, 16), 1),
        16)
    packed = jnp.bitwise_or(rne[:, :_WORDS],
                            jax.lax.shift_left(rne[:, _WORDS:], 16))
    w_flat = jax.lax.bitcast_convert_type(packed, jnp.float32).reshape(-1)
    out = _bow_sc(ids_flat, w_flat)
    return out.reshape(_NUM_CLASSES, _BATCH).T + b.astype(jnp.float32)


# speedup vs baseline: 1.0033x; 1.0033x over previous
import functools

import jax
import jax.numpy as jnp
from jax import lax
from jax.experimental import pallas as pl
from jax.experimental.pallas import tpu as pltpu
from jax.experimental.pallas import tpu_sc as plsc

_VOCAB = 100000
_NUM_CLASSES = 2
_BATCH = 1024
_SEQ = 200
_N_SUBCORES = 16
_ROWS_PER = _BATCH // _N_SUBCORES
_IDS_PER = _ROWS_PER * _SEQ
_LANES = 16
_GROUPS = _ROWS_PER // _LANES
_WORDS = _VOCAB // 2
_N_CHUNKS = 10
_CHUNK = _WORDS // _N_CHUNKS


def _bow_body(ids_hbm, w_hbm, out_hbm, table_v, ids_v, out_v, sem_w, sem_i):
    cls = lax.axis_index("c")
    sub = lax.axis_index("s")
    rowbase = sub * _ROWS_PER

    ids_off = pl.multiple_of(sub * _IDS_PER, 8)
    cp_i = pltpu.async_copy(ids_hbm.at[pl.ds(ids_off, _IDS_PER)], ids_v, sem_i)

    w_base = pl.multiple_of(cls * _WORDS, 8)
    start = lax.rem(sub, _N_CHUNKS)
    cps = []
    for k in range(_N_CHUNKS):
        sl = lax.rem(start + k, _N_CHUNKS) * _CHUNK
        sl = pl.multiple_of(sl, 8)
        cps.append(pltpu.async_copy(w_hbm.at[pl.ds(w_base + sl, _CHUNK)],
                                    table_v.at[pl.ds(sl, _CHUNK)], sem_w))
    for cp in cps:
        cp.wait()
    cp_i.wait()

    lane = lax.iota(jnp.int32, _LANES)
    head = plsc.bitcast(table_v[pl.ds(0, _LANES)], jnp.int32)
    patched = jnp.bitwise_and(head, jnp.int32(-65536))
    table_v[pl.ds(0, _LANES)] = plsc.bitcast(
        jnp.where(lane == 0, patched, head), jnp.float32)

    bases = [(g * _LANES + lane) * _SEQ for g in range(_GROUPS)]
    zero = jnp.zeros((_LANES,), jnp.float32)

    def extract(tid):
        hi = tid >= _WORDS
        widx = jnp.where(hi, tid - _WORDS, tid)
        word = plsc.bitcast(plsc.load_gather(table_v, [widx]), jnp.int32)
        sh = jnp.where(hi, jnp.int32(16), jnp.int32(0))
        bits = lax.shift_left(lax.shift_right_logical(word, sh), 16)
        return plsc.bitcast(bits, jnp.float32)

    def step(t, accs):
        ids16 = [plsc.load_gather(ids_v, [bases[g] + t])
                 for g in range(_GROUPS)]
        vals = [extract(ix) for ix in ids16]
        return tuple(accs[g] + vals[g] for g in range(_GROUPS))

    accs = lax.fori_loop(0, _SEQ, step, (zero,) * _GROUPS)
    for g in range(_GROUPS):
        out_v[pl.ds(g * _LANES, _LANES)] = accs[g]

    out_off = pl.multiple_of(cls * _BATCH + rowbase, 8)
    pltpu.sync_copy(out_v, out_hbm.at[pl.ds(out_off, _ROWS_PER)])


@jax.jit
def _bow_sc(ids_flat, w_flat):
    mesh = plsc.VectorSubcoreMesh(core_axis_name="c", subcore_axis_name="s")
    f = functools.partial(
        pl.kernel,
        mesh=mesh,
        compiler_params=pltpu.CompilerParams(needs_layout_passes=False),
        out_type=jax.ShapeDtypeStruct((_NUM_CLASSES * _BATCH,), jnp.float32),
        scratch_types=[
            pltpu.VMEM((_WORDS,), jnp.float32),
            pltpu.VMEM((_IDS_PER,), jnp.int32),
            pltpu.VMEM((_ROWS_PER,), jnp.float32),
            pltpu.SemaphoreType.DMA,
            pltpu.SemaphoreType.DMA,
        ],
    )(_bow_body)
    return f(ids_flat, w_flat)


def kernel(input_ids, W, b):
    ids_flat = input_ids.astype(jnp.int32).reshape(-1)
    w32 = jax.lax.bitcast_convert_type(W.astype(jnp.float32), jnp.int32)
    rne = jax.lax.shift_right_logical(
        w32 + 0x7FFF + jnp.bitwise_and(jax.lax.shift_right_logical(w32, 16), 1),
        16)
    packed = jnp.bitwise_or(rne[:, :_WORDS],
                            jax.lax.shift_left(rne[:, _WORDS:], 16))
    w_flat = jax.lax.bitcast_convert_type(packed, jnp.float32).reshape(-1)
    out = _bow_sc(ids_flat, w_flat)
    return out.reshape(_NUM_CLASSES, _BATCH).T + b.astype(jnp.float32)
